# Initial kernel scaffold; baseline (speedup 1.0000x reference)
import jax
import jax.numpy as jnp
from jax.experimental import pallas as pl


def kernel(x, edge_index, edge_weight, W1, b1, W2, b2, fc_W, fc_b):
    raise NotImplementedError("write your pallas kernel here")


# speedup vs baseline: 14.8247x; 14.8247x over previous
import jax
import jax.numpy as jnp
from jax import lax
from jax.experimental import pallas as pl
from jax.experimental.pallas import tpu as pltpu
from jax.experimental.pallas import tpu_sc as plsc

N = 10000
E = 160000
D = 256
H = 128
NC = 2
NT = 16
L = 16
EC = E // NT
CK = 80
ROWS = EC // CK
NPT = N // NT
BN = 2000
GB = N // BN
SHIFT = 14
MASK = (1 << SHIFT) - 1


def _mm_body(x_ref, w_ref, glo_ref, ghi_ref):
    xb = x_ref[...]
    glo_ref[...] = jnp.dot(xb, w_ref[:, :H], preferred_element_type=jnp.float32)
    ghi_ref[...] = jnp.dot(xb, w_ref[:, H:], preferred_element_type=jnp.float32)


def _matmul(x, W1):
    return pl.pallas_call(
        _mm_body,
        grid=(GB,),
        in_specs=[
            pl.BlockSpec((BN, D), lambda i: (i, 0)),
            pl.BlockSpec((D, D), lambda i: (0, 0)),
        ],
        out_specs=[pl.BlockSpec((BN, H), lambda i: (i, 0))] * 2,
        out_shape=[jax.ShapeDtypeStruct((N, H), jnp.float32)] * 2,
    )(x, W1)


def _sc_body(idx_hbm, w_hbm, glo_hbm, ghi_hbm, fcw_hbm,
             hlo_out, hhi_out, v_out, dinv_out,
             idx1, norm1, nodef, sidx, didx, vbuf, gbuf,
             acc_sp, deg_sp, v_sp, sem):
    c = lax.axis_index("c")
    s = lax.axis_index("s")
    z16 = jnp.zeros((L,), jnp.float32)

    pltpu.sync_copy(idx_hbm.at[pl.ds(s * EC, EC)], idx1)
    pltpu.sync_copy(w_hbm.at[pl.ds(s * EC, EC)], norm1)

    def _zero_nodef(j, _):
        nodef[pl.ds(j * L, L)] = z16
        return 0
    lax.fori_loop(0, N // L, _zero_nodef, 0)

    def _zero_gbuf(e, _):
        for f in range(H // L):
            gbuf[e, pl.ds(f * L, L)] = z16
        return 0
    lax.fori_loop(0, CK, _zero_gbuf, 0)

    @pl.when(s == 0)
    def _():
        pltpu.sync_copy(nodef, deg_sp)
        pltpu.sync_copy(nodef, v_sp)

    for r in range(NPT // CK):
        pltpu.sync_copy(gbuf, acc_sp.at[pl.ds(s * NPT + r * CK, CK)])
    pltpu.sync_copy(gbuf.at[pl.ds(0, NPT % CK)],
                    acc_sp.at[pl.ds(s * NPT + NPT - NPT % CK, NPT % CK)])
    plsc.subcore_barrier()

    def _unpack_dst(j):
        for k in range(CK // L):
            p16 = idx1[pl.ds(j * CK + k * L, L)]
            didx[pl.ds(k * L, L)] = p16 & MASK

    def _unpack_src(j):
        for k in range(CK // L):
            p16 = idx1[pl.ds(j * CK + k * L, L)]
            sidx[pl.ds(k * L, L)] = lax.shift_right_logical(p16, SHIFT)

    def _deg(j, _):
        _unpack_dst(j)
        pltpu.sync_copy(norm1.at[pl.ds(j * CK, CK)], deg_sp.at[didx], add=True)
        return 0
    lax.fori_loop(0, ROWS, _deg, 0)
    plsc.subcore_barrier()

    pltpu.sync_copy(deg_sp, nodef)

    def _dinv(j, _):
        d = nodef[pl.ds(j * L, L)] + 1.0
        i0 = jnp.int32(0x5F3759DF) - lax.shift_right_logical(
            lax.bitcast_convert_type(d, jnp.int32), 1)
        y = lax.bitcast_convert_type(i0, jnp.float32)
        y = y * (1.5 - 0.5 * d * y * y)
        y = y * (1.5 - 0.5 * d * y * y)
        y = y * (1.5 - 0.5 * d * y * y)
        nodef[pl.ds(j * L, L)] = y
        return 0
    lax.fori_loop(0, N // L, _dinv, 0)

    @pl.when(jnp.logical_and(c == 0, s == 0))
    def _():
        pltpu.sync_copy(nodef, dinv_out)

    def _edges(g_hbm):
        def _chunk(j, _):
            _unpack_src(j)
            _unpack_dst(j)
            cp = pltpu.async_copy(g_hbm.at[sidx], gbuf, sem)
            for k in range(CK // L):
                s16 = sidx[pl.ds(k * L, L)]
                d16 = didx[pl.ds(k * L, L)]
                w16 = norm1[pl.ds(j * CK + k * L, L)]
                dv_s = plsc.load_gather(nodef, [s16])
                dv_d = plsc.load_gather(nodef, [d16])
                norm1[pl.ds(j * CK + k * L, L)] = dv_s * w16 * dv_d
            cp.wait()

            def _row16(k, _):
                n16 = norm1[pl.ds(j * CK + k * L, L)]
                for r in range(L):
                    e = k * L + r
                    n = n16[r]
                    for f in range(H // L):
                        gbuf[e, pl.ds(f * L, L)] = gbuf[e, pl.ds(f * L, L)] * n
                return 0
            lax.fori_loop(0, CK // L, _row16, 0)
            pltpu.sync_copy(gbuf, acc_sp.at[didx], add=True)
            return 0
        lax.fori_loop(0, ROWS, _chunk, 0)

    @pl.when(c == 0)
    def _():
        _edges(glo_hbm)

    @pl.when(c == 1)
    def _():
        _edges(ghi_hbm)
    plsc.subcore_barrier()

    @pl.when(c == 0)
    def _():
        pltpu.sync_copy(acc_sp.at[pl.ds(s * NPT, NPT)], hlo_out.at[s])

    @pl.when(c == 1)
    def _():
        pltpu.sync_copy(acc_sp.at[pl.ds(s * NPT, NPT)], hhi_out.at[s])

    @pl.when(c == 0)
    def _():
        pltpu.sync_copy(fcw_hbm, nodef)

        def _vscat(j, _):
            _unpack_src(j)
            _unpack_dst(j)
            for k in range(CK // L):
                d16 = didx[pl.ds(k * L, L)]
                n16 = norm1[pl.ds(j * CK + k * L, L)]
                fw = plsc.load_gather(nodef, [d16])
                vbuf[pl.ds(k * L, L)] = n16 * fw
            pltpu.sync_copy(vbuf, v_sp.at[sidx], add=True)
            return 0
        lax.fori_loop(0, ROWS, _vscat, 0)
    plsc.subcore_barrier()

    @pl.when(jnp.logical_and(c == 0, s == 0))
    def _():
        pltpu.sync_copy(v_sp, v_out)


def _sc_call(idx2, w2, g_lo, g_hi, fcw):
    mesh = plsc.VectorSubcoreMesh(core_axis_name="c", subcore_axis_name="s")
    f = pl.kernel(
        _sc_body,
        out_type=[
            jax.ShapeDtypeStruct((NT, NPT, H), jnp.float32),
            jax.ShapeDtypeStruct((NT, NPT, H), jnp.float32),
            jax.ShapeDtypeStruct((N,), jnp.float32),
            jax.ShapeDtypeStruct((N,), jnp.float32),
        ],
        mesh=mesh,
        compiler_params=pltpu.CompilerParams(needs_layout_passes=False),
        scratch_types=[
            pltpu.VMEM((EC,), jnp.int32),
            pltpu.VMEM((EC,), jnp.float32),
            pltpu.VMEM((N,), jnp.float32),
            pltpu.VMEM((CK,), jnp.int32),
            pltpu.VMEM((CK,), jnp.int32),
            pltpu.VMEM((CK,), jnp.float32),
            pltpu.VMEM((CK, H), jnp.float32),
            pltpu.VMEM_SHARED((N, H), jnp.float32),
            pltpu.VMEM_SHARED((N,), jnp.float32),
            pltpu.VMEM_SHARED((N,), jnp.float32),
            pltpu.SemaphoreType.DMA,
        ],
    )
    return f(idx2, w2, g_lo, g_hi, fcw)


def _red_body(hlo_ref, hhi_ref, glo_ref, ghi_ref, dinv_ref, v_ref,
              fcw_ref, b1_ref, w2_ref, b2_ref, fcb_ref, out_ref, acc, sfc):
    i = pl.program_id(0)

    @pl.when(i == 0)
    def _():
        acc[...] = jnp.zeros((2, H), jnp.float32)
        sfc[0] = 0.0

    dv = dinv_ref[0, 0]
    dv2 = dv * dv
    fw = fcw_ref[0, 0]
    vf = (v_ref[0, 0] + dv2 * fw)[None, :]
    for q, (h_ref, g_ref) in enumerate(((hlo_ref, glo_ref), (hhi_ref, ghi_ref))):
        hq = jnp.maximum(
            h_ref[...] + dv2[:, None] * g_ref[...] + b1_ref[q][None, :], 0.0)
        acc[q:q + 1, :] = acc[q:q + 1, :] + jnp.dot(
            vf, hq, preferred_element_type=jnp.float32)
    sfc[0] = sfc[0] + jnp.sum(fw)

    @pl.when(i == GB - 1)
    def _():
        total = (jnp.sum(acc[...] * w2_ref[...])
                 + b2_ref[0, 0] * sfc[0] + fcb_ref[0, 0])
        out_ref[...] = jnp.reshape(total, (1, 1))


def _reduce(hlo, hhi, g_lo, g_hi, dinv, v, fcw, b1, W2, b2, fc_b):
    dinv2d = dinv.reshape(GB, 1, BN)
    v2d = v.reshape(GB, 1, BN)
    fcw2d = fcw.reshape(GB, 1, BN)
    b1r = b1.reshape(2, H)
    w2r = W2[:, 0].reshape(2, H)
    b2r = b2.reshape(1, 1)
    fcbr = fc_b.reshape(1, 1)
    out = pl.pallas_call(
        _red_body,
        grid=(GB,),
        in_specs=(
            [pl.BlockSpec((BN, H), lambda i: (i, 0))] * 4 + [
                pl.BlockSpec((1, 1, BN), lambda i: (i, 0, 0)),
                pl.BlockSpec((1, 1, BN), lambda i: (i, 0, 0)),
                pl.BlockSpec((1, 1, BN), lambda i: (i, 0, 0)),
                pl.BlockSpec((2, H), lambda i: (0, 0)),
                pl.BlockSpec((2, H), lambda i: (0, 0)),
                pl.BlockSpec((1, 1), lambda i: (0, 0)),
                pl.BlockSpec((1, 1), lambda i: (0, 0)),
            ]
        ),
        out_specs=pl.BlockSpec((1, 1), lambda i: (0, 0)),
        out_shape=jax.ShapeDtypeStruct((1, 1), jnp.float32),
        scratch_shapes=[
            pltpu.VMEM((2, H), jnp.float32),
            pltpu.SMEM((1,), jnp.float32),
        ],
    )(hlo, hhi, g_lo, g_hi, dinv2d, v2d, fcw2d, b1r, w2r, b2r, fcbr)
    return out.reshape(1)


def kernel(x, edge_index, edge_weight, W1, b1, W2, b2, fc_W, fc_b):
    ei = edge_index.astype(jnp.int32)
    packed = jnp.bitwise_or(jnp.left_shift(ei[0], SHIFT), ei[1])
    idx2 = packed
    w2 = edge_weight
    fcw = fc_W[:, 0]

    g_lo, g_hi = _matmul(x, W1)
    hlo, hhi, v, dinv = _sc_call(idx2, w2, g_lo, g_hi, fcw)
    return _reduce(hlo.reshape(N, H), hhi.reshape(N, H), g_lo, g_hi,
                   dinv, v, fcw, b1, W2, b2, fc_b)


# speedup vs baseline: 24.5774x; 1.6579x over previous
import jax
import jax.numpy as jnp
from jax import lax
from jax.experimental import pallas as pl
from jax.experimental.pallas import tpu as pltpu
from jax.experimental.pallas import tpu_sc as plsc

N = 10000
E = 160000
D = 256
H = 128
NC = 2
NT = 16
L = 16
EC = E // NT
CK = 80
ROWS = EC // CK
NPT = N // NT
BN = 2000
GB = N // BN
SHIFT = 14
MASK = (1 << SHIFT) - 1


def _mm_body(x_ref, w_ref, glo_ref, ghi_ref):
    xb = x_ref[...]
    glo_ref[...] = jnp.dot(xb, w_ref[:, :H], preferred_element_type=jnp.float32)
    ghi_ref[...] = jnp.dot(xb, w_ref[:, H:], preferred_element_type=jnp.float32)


def _matmul(x, W1):
    return pl.pallas_call(
        _mm_body,
        grid=(GB,),
        in_specs=[
            pl.BlockSpec((BN, D), lambda i: (i, 0)),
            pl.BlockSpec((D, D), lambda i: (0, 0)),
        ],
        out_specs=[pl.BlockSpec((BN, H), lambda i: (i, 0))] * 2,
        out_shape=[jax.ShapeDtypeStruct((N, H), jnp.float32)] * 2,
    )(x, W1)


NROW = (N + H - 1) // H


def _sc_body(idx_hbm, w_hbm, glo_hbm, ghi_hbm, fcw_hbm,
             hlo_out, hhi_out, v_out, dinv_out,
             idx1, norm1, sidxA, didxA, sidxB, didxB, rows80,
             gbuf, gbuf2, acc_sp, deg_sp, v_sp,
             gsemA, gsemB, ssemA, ssemB):
    c = lax.axis_index("c")
    s = lax.axis_index("s")
    z16 = jnp.zeros((L,), jnp.float32)
    iota16 = lax.iota(jnp.int32, L)

    pltpu.sync_copy(idx_hbm.at[pl.ds(s * EC, EC)], idx1)
    pltpu.sync_copy(w_hbm.at[pl.ds(s * EC, EC)], norm1)

    def _zero2d(buf):
        def _z(r, _):
            for f in range(H // L):
                buf[r, pl.ds(f * L, L)] = z16
            return 0
        lax.fori_loop(0, CK, _z, 0)

    _zero2d(gbuf)
    _zero2d(gbuf2)
    for k in range(CK // L):
        rows80[pl.ds(k * L, L)] = iota16 + k * L

    @pl.when(s == 0)
    def _():
        pltpu.sync_copy(gbuf, deg_sp)
        pltpu.sync_copy(gbuf, v_sp)

    for r in range(NPT // CK):
        pltpu.sync_copy(gbuf, acc_sp.at[pl.ds(s * NPT + r * CK, CK)])
    pltpu.sync_copy(gbuf.at[pl.ds(0, NPT % CK)],
                    acc_sp.at[pl.ds(s * NPT + NPT - NPT % CK, NPT % CK)])
    plsc.subcore_barrier()

    def _deg(j, _):
        for k in range(CK // L):
            p16 = idx1[pl.ds(j * CK + k * L, L)]
            d16 = p16 & MASK
            w16 = norm1[pl.ds(j * CK + k * L, L)]
            plsc.addupdate_scatter(
                gbuf, [lax.shift_right_logical(d16, 7), d16 & (H - 1)], w16)
        return 0
    lax.fori_loop(0, ROWS, _deg, 0)
    pltpu.sync_copy(gbuf, deg_sp.at[rows80], add=True)
    plsc.subcore_barrier()

    pltpu.sync_copy(deg_sp, gbuf)

    def _dinv(r, _):
        for f in range(H // L):
            d = gbuf[r, pl.ds(f * L, L)] + 1.0
            i0 = jnp.int32(0x5F3759DF) - lax.shift_right_logical(
                lax.bitcast_convert_type(d, jnp.int32), 1)
            y = lax.bitcast_convert_type(i0, jnp.float32)
            y = y * (1.5 - 0.5 * d * y * y)
            y = y * (1.5 - 0.5 * d * y * y)
            y = y * (1.5 - 0.5 * d * y * y)
            gbuf[r, pl.ds(f * L, L)] = y
        return 0
    lax.fori_loop(0, CK, _dinv, 0)

    @pl.when(jnp.logical_and(c == 0, s == 0))
    def _():
        pltpu.sync_copy(gbuf, dinv_out)

    def _norm(j, _):
        for k in range(CK // L):
            sl = pl.ds(j * CK + k * L, L)
            p16 = idx1[sl]
            s16 = lax.shift_right_logical(p16, SHIFT)
            d16 = p16 & MASK
            dv_s = plsc.load_gather(
                gbuf, [lax.shift_right_logical(s16, 7), s16 & (H - 1)])
            dv_d = plsc.load_gather(
                gbuf, [lax.shift_right_logical(d16, 7), d16 & (H - 1)])
            norm1[sl] = dv_s * norm1[sl] * dv_d
        return 0
    lax.fori_loop(0, ROWS, _norm, 0)

    def _unpack(j, si, di):
        for k in range(CK // L):
            p16 = idx1[pl.ds(j * CK + k * L, L)]
            si[pl.ds(k * L, L)] = lax.shift_right_logical(p16, SHIFT)
            di[pl.ds(k * L, L)] = p16 & MASK

    def _scale(j, gb):
        def _row16(k, _):
            n16 = norm1[pl.ds(j * CK + k * L, L)]
            for r in range(L):
                e = k * L + r
                n = n16[r]
                for f in range(H // L):
                    gb[e, pl.ds(f * L, L)] = gb[e, pl.ds(f * L, L)] * n
            return 0
        lax.fori_loop(0, CK // L, _row16, 0)

    def _edges(g_hbm):
        _unpack(0, sidxA, didxA)
        pltpu.async_copy(g_hbm.at[sidxA], gbuf, gsemA)

        def _iter(j, own, oth):
            gb, si, di, gsem, ssem = own
            gbo, sio, dio, gsemo, ssemo = oth

            @pl.when(j + 1 < ROWS)
            def _():
                @pl.when(j >= 1)
                def _():
                    pltpu.make_async_copy(gbo, acc_sp.at[dio], ssemo).wait()
                _unpack(j + 1, sio, dio)
                pltpu.async_copy(g_hbm.at[sio], gbo, gsemo)

            pltpu.make_async_copy(g_hbm.at[si], gb, gsem).wait()
            _scale(j, gb)
            pltpu.async_copy(gb, acc_sp.at[di], ssem, add=True)

        A = (gbuf, sidxA, didxA, gsemA, ssemA)
        B = (gbuf2, sidxB, didxB, gsemB, ssemB)

        def _chunk(j, _):
            @pl.when(j % 2 == 0)
            def _():
                _iter(j, A, B)

            @pl.when(j % 2 == 1)
            def _():
                _iter(j, B, A)
            return 0
        lax.fori_loop(0, ROWS, _chunk, 0)
        pltpu.make_async_copy(gbuf2, acc_sp.at[didxB], ssemB).wait()
        pltpu.make_async_copy(gbuf, acc_sp.at[didxA], ssemA).wait()

    @pl.when(c == 0)
    def _():
        _edges(glo_hbm)

    @pl.when(c == 1)
    def _():
        _edges(ghi_hbm)
    plsc.subcore_barrier()

    @pl.when(c == 0)
    def _():
        pltpu.sync_copy(acc_sp.at[pl.ds(s * NPT, NPT)], hlo_out.at[s])

    @pl.when(c == 1)
    def _():
        pltpu.sync_copy(acc_sp.at[pl.ds(s * NPT, NPT)], hhi_out.at[s])

    @pl.when(c == 0)
    def _():
        pltpu.sync_copy(fcw_hbm, gbuf)
        _zero2d(gbuf2)

        def _vscat(j, _):
            for k in range(CK // L):
                sl = pl.ds(j * CK + k * L, L)
                p16 = idx1[sl]
                s16 = lax.shift_right_logical(p16, SHIFT)
                d16 = p16 & MASK
                fw = plsc.load_gather(
                    gbuf, [lax.shift_right_logical(d16, 7), d16 & (H - 1)])
                plsc.addupdate_scatter(
                    gbuf2, [lax.shift_right_logical(s16, 7), s16 & (H - 1)],
                    norm1[sl] * fw)
            return 0
        lax.fori_loop(0, ROWS, _vscat, 0)
        pltpu.sync_copy(gbuf2, v_sp.at[rows80], add=True)
    plsc.subcore_barrier()

    @pl.when(jnp.logical_and(c == 0, s == 0))
    def _():
        pltpu.sync_copy(v_sp, v_out)


def _sc_call(idx2, w2, g_lo, g_hi, fcw2):
    mesh = plsc.VectorSubcoreMesh(core_axis_name="c", subcore_axis_name="s")
    f = pl.kernel(
        _sc_body,
        out_type=[
            jax.ShapeDtypeStruct((NT, NPT, H), jnp.float32),
            jax.ShapeDtypeStruct((NT, NPT, H), jnp.float32),
            jax.ShapeDtypeStruct((CK, H), jnp.float32),
            jax.ShapeDtypeStruct((CK, H), jnp.float32),
        ],
        mesh=mesh,
        compiler_params=pltpu.CompilerParams(needs_layout_passes=False),
        scratch_types=[
            pltpu.VMEM((EC,), jnp.int32),
            pltpu.VMEM((EC,), jnp.float32),
            pltpu.VMEM((CK,), jnp.int32),
            pltpu.VMEM((CK,), jnp.int32),
            pltpu.VMEM((CK,), jnp.int32),
            pltpu.VMEM((CK,), jnp.int32),
            pltpu.VMEM((CK,), jnp.int32),
            pltpu.VMEM((CK, H), jnp.float32),
            pltpu.VMEM((CK, H), jnp.float32),
            pltpu.VMEM_SHARED((N, H), jnp.float32),
            pltpu.VMEM_SHARED((CK, H), jnp.float32),
            pltpu.VMEM_SHARED((CK, H), jnp.float32),
            pltpu.SemaphoreType.DMA,
            pltpu.SemaphoreType.DMA,
            pltpu.SemaphoreType.DMA,
            pltpu.SemaphoreType.DMA,
        ],
    )
    return f(idx2, w2, g_lo, g_hi, fcw2)


def _red_body(hlo_ref, hhi_ref, glo_ref, ghi_ref, dinv_ref, v_ref,
              fcw_ref, b1_ref, w2_ref, b2_ref, fcb_ref, out_ref, acc, sfc):
    i = pl.program_id(0)

    @pl.when(i == 0)
    def _():
        acc[...] = jnp.zeros((2, H), jnp.float32)
        sfc[0] = 0.0

    dv = dinv_ref[0, 0]
    dv2 = dv * dv
    fw = fcw_ref[0, 0]
    vf = (v_ref[0, 0] + dv2 * fw)[None, :]
    for q, (h_ref, g_ref) in enumerate(((hlo_ref, glo_ref), (hhi_ref, ghi_ref))):
        hq = jnp.maximum(
            h_ref[...] + dv2[:, None] * g_ref[...] + b1_ref[q][None, :], 0.0)
        acc[q:q + 1, :] = acc[q:q + 1, :] + jnp.dot(
            vf, hq, preferred_element_type=jnp.float32)
    sfc[0] = sfc[0] + jnp.sum(fw)

    @pl.when(i == GB - 1)
    def _():
        total = (jnp.sum(acc[...] * w2_ref[...])
                 + b2_ref[0, 0] * sfc[0] + fcb_ref[0, 0])
        out_ref[...] = jnp.reshape(total, (1, 1))


def _reduce(hlo, hhi, g_lo, g_hi, dinv, v, fcw, b1, W2, b2, fc_b):
    dinv2d = dinv.reshape(GB, 1, BN)
    v2d = v.reshape(GB, 1, BN)
    fcw2d = fcw.reshape(GB, 1, BN)
    b1r = b1.reshape(2, H)
    w2r = W2[:, 0].reshape(2, H)
    b2r = b2.reshape(1, 1)
    fcbr = fc_b.reshape(1, 1)
    out = pl.pallas_call(
        _red_body,
        grid=(GB,),
        in_specs=(
            [pl.BlockSpec((BN, H), lambda i: (i, 0))] * 4 + [
                pl.BlockSpec((1, 1, BN), lambda i: (i, 0, 0)),
                pl.BlockSpec((1, 1, BN), lambda i: (i, 0, 0)),
                pl.BlockSpec((1, 1, BN), lambda i: (i, 0, 0)),
                pl.BlockSpec((2, H), lambda i: (0, 0)),
                pl.BlockSpec((2, H), lambda i: (0, 0)),
                pl.BlockSpec((1, 1), lambda i: (0, 0)),
                pl.BlockSpec((1, 1), lambda i: (0, 0)),
            ]
        ),
        out_specs=pl.BlockSpec((1, 1), lambda i: (0, 0)),
        out_shape=jax.ShapeDtypeStruct((1, 1), jnp.float32),
        scratch_shapes=[
            pltpu.VMEM((2, H), jnp.float32),
            pltpu.SMEM((1,), jnp.float32),
        ],
    )(hlo, hhi, g_lo, g_hi, dinv2d, v2d, fcw2d, b1r, w2r, b2r, fcbr)
    return out.reshape(1)


def kernel(x, edge_index, edge_weight, W1, b1, W2, b2, fc_W, fc_b):
    ei = edge_index.astype(jnp.int32)
    packed = jnp.bitwise_or(jnp.left_shift(ei[0], SHIFT), ei[1])
    idx2 = packed
    w2 = edge_weight
    fcw = fc_W[:, 0]
    fcw2 = jnp.pad(fcw, (0, CK * H - N)).reshape(CK, H)

    g_lo, g_hi = _matmul(x, W1)
    hlo, hhi, v2, dinv2 = _sc_call(idx2, w2, g_lo, g_hi, fcw2)
    v = v2.reshape(CK * H)[:N]
    dinv = dinv2.reshape(CK * H)[:N]
    return _reduce(hlo.reshape(N, H), hhi.reshape(N, H), g_lo, g_hi,
                   dinv, v, fcw, b1, W2, b2, fc_b)


# speedup vs baseline: 24.6699x; 1.0038x over previous
import jax
import jax.numpy as jnp
from jax import lax
from jax.experimental import pallas as pl
from jax.experimental.pallas import tpu as pltpu
from jax.experimental.pallas import tpu_sc as plsc

N = 10000
E = 160000
D = 256
H = 128
NC = 2
NT = 16
L = 16
EC = E // NT
CK = 80
ROWS = EC // CK
NPT = N // NT
BN = 2000
GB = N // BN
SHIFT = 14
MASK = (1 << SHIFT) - 1


def _mm_body(x_ref, w_ref, glo_ref, ghi_ref):
    xb = x_ref[...]
    glo_ref[...] = jnp.dot(xb, w_ref[:, :H], preferred_element_type=jnp.float32)
    ghi_ref[...] = jnp.dot(xb, w_ref[:, H:], preferred_element_type=jnp.float32)


def _matmul(x, W1):
    return pl.pallas_call(
        _mm_body,
        grid=(GB,),
        in_specs=[
            pl.BlockSpec((BN, D), lambda i: (i, 0)),
            pl.BlockSpec((D, D), lambda i: (0, 0)),
        ],
        out_specs=[pl.BlockSpec((BN, H), lambda i: (i, 0))] * 2,
        out_shape=[jax.ShapeDtypeStruct((N, H), jnp.float32)] * 2,
    )(x, W1)


NROW = (N + H - 1) // H


def _sc_body(idx_hbm, w_hbm, glo_hbm, ghi_hbm, fcw_hbm,
             hlo_out, hhi_out, v_out, dinv_out,
             idx1, norm1, sidxA, didxA, sidxB, didxB, rows80,
             gbuf, gbuf2, acc_sp, deg_sp, v_sp,
             gsemA, gsemB, gsem2A, gsem2B, ssemA, ssemB):
    c = lax.axis_index("c")
    s = lax.axis_index("s")
    z16 = jnp.zeros((L,), jnp.float32)
    iota16 = lax.iota(jnp.int32, L)

    pltpu.sync_copy(idx_hbm.at[pl.ds(s * EC, EC)], idx1)
    pltpu.sync_copy(w_hbm.at[pl.ds(s * EC, EC)], norm1)

    def _zero2d(buf):
        def _z(r, _):
            for f in range(H // L):
                buf[r, pl.ds(f * L, L)] = z16
            return 0
        lax.fori_loop(0, CK, _z, 0)

    _zero2d(gbuf)
    _zero2d(gbuf2)
    for k in range(CK // L):
        rows80[pl.ds(k * L, L)] = iota16 + k * L

    @pl.when(s == 0)
    def _():
        pltpu.sync_copy(gbuf, deg_sp)
        pltpu.sync_copy(gbuf, v_sp)

    for r in range(NPT // CK):
        pltpu.sync_copy(gbuf, acc_sp.at[pl.ds(s * NPT + r * CK, CK)])
    pltpu.sync_copy(gbuf.at[pl.ds(0, NPT % CK)],
                    acc_sp.at[pl.ds(s * NPT + NPT - NPT % CK, NPT % CK)])
    plsc.subcore_barrier()

    def _deg(j, _):
        for k in range(CK // L):
            p16 = idx1[pl.ds(j * CK + k * L, L)]
            d16 = p16 & MASK
            w16 = norm1[pl.ds(j * CK + k * L, L)]
            plsc.addupdate_scatter(
                gbuf, [lax.shift_right_logical(d16, 7), d16 & (H - 1)], w16)
        return 0

    def _deg5(j5, _):
        for u in range(5):
            _deg(j5 * 5 + u, 0)
        return 0
    lax.fori_loop(0, ROWS // 5, _deg5, 0)
    pltpu.sync_copy(gbuf, deg_sp.at[rows80], add=True)
    plsc.subcore_barrier()

    pltpu.sync_copy(deg_sp, gbuf)

    def _dinv(r, _):
        for f in range(H // L):
            d = gbuf[r, pl.ds(f * L, L)] + 1.0
            i0 = jnp.int32(0x5F3759DF) - lax.shift_right_logical(
                lax.bitcast_convert_type(d, jnp.int32), 1)
            y = lax.bitcast_convert_type(i0, jnp.float32)
            y = y * (1.5 - 0.5 * d * y * y)
            y = y * (1.5 - 0.5 * d * y * y)
            y = y * (1.5 - 0.5 * d * y * y)
            gbuf[r, pl.ds(f * L, L)] = y
        return 0
    lax.fori_loop(0, CK, _dinv, 0)

    @pl.when(jnp.logical_and(c == 0, s == 0))
    def _():
        pltpu.sync_copy(gbuf, dinv_out)

    def _norm(j, _):
        for k in range(CK // L):
            sl = pl.ds(j * CK + k * L, L)
            p16 = idx1[sl]
            s16 = lax.shift_right_logical(p16, SHIFT)
            d16 = p16 & MASK
            dv_s = plsc.load_gather(
                gbuf, [lax.shift_right_logical(s16, 7), s16 & (H - 1)])
            dv_d = plsc.load_gather(
                gbuf, [lax.shift_right_logical(d16, 7), d16 & (H - 1)])
            norm1[sl] = dv_s * norm1[sl] * dv_d
        return 0

    def _norm5(j5, _):
        for u in range(5):
            _norm(j5 * 5 + u, 0)
        return 0
    lax.fori_loop(0, ROWS // 5, _norm5, 0)

    def _unpack(j, si, di):
        for k in range(CK // L):
            p16 = idx1[pl.ds(j * CK + k * L, L)]
            si[pl.ds(k * L, L)] = lax.shift_right_logical(p16, SHIFT)
            di[pl.ds(k * L, L)] = p16 & MASK

    def _scale(j, gb, k0, k1):
        def _row16(k, _):
            n16 = norm1[pl.ds(j * CK + k * L, L)]
            for r in range(L):
                e = k * L + r
                n = n16[r]
                for f in range(H // L):
                    gb[e, pl.ds(f * L, L)] = gb[e, pl.ds(f * L, L)] * n
            return 0
        lax.fori_loop(k0, k1, _row16, 0)

    GS0 = 3 * L

    def _edges(g_hbm):
        _unpack(0, sidxA, didxA)
        pltpu.async_copy(g_hbm.at[sidxA.at[pl.ds(0, GS0)]],
                         gbuf.at[pl.ds(0, GS0)], gsemA)
        pltpu.async_copy(g_hbm.at[sidxA.at[pl.ds(GS0, CK - GS0)]],
                         gbuf.at[pl.ds(GS0, CK - GS0)], gsem2A)

        def _iter(j, own, oth):
            gb, si, di, gsem, gsem2, ssem = own
            gbo, sio, dio, gsemo, gsem2o, ssemo = oth

            @pl.when(j + 1 < ROWS)
            def _():
                @pl.when(j >= 1)
                def _():
                    pltpu.make_async_copy(gbo, acc_sp.at[dio], ssemo).wait()
                _unpack(j + 1, sio, dio)
                pltpu.async_copy(g_hbm.at[sio.at[pl.ds(0, GS0)]],
                                 gbo.at[pl.ds(0, GS0)], gsemo)
                pltpu.async_copy(g_hbm.at[sio.at[pl.ds(GS0, CK - GS0)]],
                                 gbo.at[pl.ds(GS0, CK - GS0)], gsem2o)

            pltpu.make_async_copy(g_hbm.at[si.at[pl.ds(0, GS0)]],
                                  gb.at[pl.ds(0, GS0)], gsem).wait()
            _scale(j, gb, 0, GS0 // L)
            pltpu.make_async_copy(g_hbm.at[si.at[pl.ds(GS0, CK - GS0)]],
                                  gb.at[pl.ds(GS0, CK - GS0)], gsem2).wait()
            _scale(j, gb, GS0 // L, CK // L)
            pltpu.async_copy(gb, acc_sp.at[di], ssem, add=True)

        A = (gbuf, sidxA, didxA, gsemA, gsem2A, ssemA)
        B = (gbuf2, sidxB, didxB, gsemB, gsem2B, ssemB)

        def _chunk(j, _):
            @pl.when(j % 2 == 0)
            def _():
                _iter(j, A, B)

            @pl.when(j % 2 == 1)
            def _():
                _iter(j, B, A)
            return 0
        lax.fori_loop(0, ROWS, _chunk, 0)
        pltpu.make_async_copy(gbuf2, acc_sp.at[didxB], ssemB).wait()
        pltpu.make_async_copy(gbuf, acc_sp.at[didxA], ssemA).wait()

    @pl.when(c == 0)
    def _():
        _edges(glo_hbm)

    @pl.when(c == 1)
    def _():
        _edges(ghi_hbm)
    plsc.subcore_barrier()

    @pl.when(c == 0)
    def _():
        pltpu.sync_copy(acc_sp.at[pl.ds(s * NPT, NPT)], hlo_out.at[s])

    @pl.when(c == 1)
    def _():
        pltpu.sync_copy(acc_sp.at[pl.ds(s * NPT, NPT)], hhi_out.at[s])

    @pl.when(c == 0)
    def _():
        pltpu.sync_copy(fcw_hbm, gbuf)
        _zero2d(gbuf2)

        def _vscat(j, _):
            for k in range(CK // L):
                sl = pl.ds(j * CK + k * L, L)
                p16 = idx1[sl]
                s16 = lax.shift_right_logical(p16, SHIFT)
                d16 = p16 & MASK
                fw = plsc.load_gather(
                    gbuf, [lax.shift_right_logical(d16, 7), d16 & (H - 1)])
                plsc.addupdate_scatter(
                    gbuf2, [lax.shift_right_logical(s16, 7), s16 & (H - 1)],
                    norm1[sl] * fw)
            return 0

        def _vscat5(j5, _):
            for u in range(5):
                _vscat(j5 * 5 + u, 0)
            return 0
        lax.fori_loop(0, ROWS // 5, _vscat5, 0)
        pltpu.sync_copy(gbuf2, v_sp.at[rows80], add=True)
    plsc.subcore_barrier()

    @pl.when(jnp.logical_and(c == 0, s == 0))
    def _():
        pltpu.sync_copy(v_sp, v_out)


def _sc_call(idx2, w2, g_lo, g_hi, fcw2):
    mesh = plsc.VectorSubcoreMesh(core_axis_name="c", subcore_axis_name="s")
    f = pl.kernel(
        _sc_body,
        out_type=[
            jax.ShapeDtypeStruct((NT, NPT, H), jnp.float32),
            jax.ShapeDtypeStruct((NT, NPT, H), jnp.float32),
            jax.ShapeDtypeStruct((CK, H), jnp.float32),
            jax.ShapeDtypeStruct((CK, H), jnp.float32),
        ],
        mesh=mesh,
        compiler_params=pltpu.CompilerParams(needs_layout_passes=False),
        scratch_types=[
            pltpu.VMEM((EC,), jnp.int32),
            pltpu.VMEM((EC,), jnp.float32),
            pltpu.VMEM((CK,), jnp.int32),
            pltpu.VMEM((CK,), jnp.int32),
            pltpu.VMEM((CK,), jnp.int32),
            pltpu.VMEM((CK,), jnp.int32),
            pltpu.VMEM((CK,), jnp.int32),
            pltpu.VMEM((CK, H), jnp.float32),
            pltpu.VMEM((CK, H), jnp.float32),
            pltpu.VMEM_SHARED((N, H), jnp.float32),
            pltpu.VMEM_SHARED((CK, H), jnp.float32),
            pltpu.VMEM_SHARED((CK, H), jnp.float32),
            pltpu.SemaphoreType.DMA,
            pltpu.SemaphoreType.DMA,
            pltpu.SemaphoreType.DMA,
            pltpu.SemaphoreType.DMA,
            pltpu.SemaphoreType.DMA,
            pltpu.SemaphoreType.DMA,
        ],
    )
    return f(idx2, w2, g_lo, g_hi, fcw2)


def _red_body(hlo_ref, hhi_ref, glo_ref, ghi_ref, dinv_ref, v_ref,
              fcw_ref, b1_ref, w2_ref, b2_ref, fcb_ref, out_ref, acc, sfc):
    i = pl.program_id(0)

    @pl.when(i == 0)
    def _():
        acc[...] = jnp.zeros((2, H), jnp.float32)
        sfc[0] = 0.0

    dv = dinv_ref[0, 0]
    dv2 = dv * dv
    fw = fcw_ref[0, 0]
    vf = (v_ref[0, 0] + dv2 * fw)[None, :]
    for q, (h_ref, g_ref) in enumerate(((hlo_ref, glo_ref), (hhi_ref, ghi_ref))):
        hq = jnp.maximum(
            h_ref[...] + dv2[:, None] * g_ref[...] + b1_ref[q][None, :], 0.0)
        acc[q:q + 1, :] = acc[q:q + 1, :] + jnp.dot(
            vf, hq, preferred_element_type=jnp.float32)
    sfc[0] = sfc[0] + jnp.sum(fw)

    @pl.when(i == GB - 1)
    def _():
        total = (jnp.sum(acc[...] * w2_ref[...])
                 + b2_ref[0, 0] * sfc[0] + fcb_ref[0, 0])
        out_ref[...] = jnp.reshape(total, (1, 1))


def _reduce(hlo, hhi, g_lo, g_hi, dinv, v, fcw, b1, W2, b2, fc_b):
    dinv2d = dinv.reshape(GB, 1, BN)
    v2d = v.reshape(GB, 1, BN)
    fcw2d = fcw.reshape(GB, 1, BN)
    b1r = b1.reshape(2, H)
    w2r = W2[:, 0].reshape(2, H)
    b2r = b2.reshape(1, 1)
    fcbr = fc_b.reshape(1, 1)
    out = pl.pallas_call(
        _red_body,
        grid=(GB,),
        in_specs=(
            [pl.BlockSpec((BN, H), lambda i: (i, 0))] * 4 + [
                pl.BlockSpec((1, 1, BN), lambda i: (i, 0, 0)),
                pl.BlockSpec((1, 1, BN), lambda i: (i, 0, 0)),
                pl.BlockSpec((1, 1, BN), lambda i: (i, 0, 0)),
                pl.BlockSpec((2, H), lambda i: (0, 0)),
                pl.BlockSpec((2, H), lambda i: (0, 0)),
                pl.BlockSpec((1, 1), lambda i: (0, 0)),
                pl.BlockSpec((1, 1), lambda i: (0, 0)),
            ]
        ),
        out_specs=pl.BlockSpec((1, 1), lambda i: (0, 0)),
        out_shape=jax.ShapeDtypeStruct((1, 1), jnp.float32),
        scratch_shapes=[
            pltpu.VMEM((2, H), jnp.float32),
            pltpu.SMEM((1,), jnp.float32),
        ],
    )(hlo, hhi, g_lo, g_hi, dinv2d, v2d, fcw2d, b1r, w2r, b2r, fcbr)
    return out.reshape(1)


def kernel(x, edge_index, edge_weight, W1, b1, W2, b2, fc_W, fc_b):
    ei = edge_index.astype(jnp.int32)
    packed = jnp.bitwise_or(jnp.left_shift(ei[0], SHIFT), ei[1])
    idx2 = packed
    w2 = edge_weight
    fcw = fc_W[:, 0]
    fcw2 = jnp.pad(fcw, (0, CK * H - N)).reshape(CK, H)

    g_lo, g_hi = _matmul(x, W1)
    hlo, hhi, v2, dinv2 = _sc_call(idx2, w2, g_lo, g_hi, fcw2)
    v = v2.reshape(CK * H)[:N]
    dinv = dinv2.reshape(CK * H)[:N]
    return _reduce(hlo.reshape(N, H), hhi.reshape(N, H), g_lo, g_hi,
                   dinv, v, fcw, b1, W2, b2, fc_b)


# speedup vs baseline: 25.0211x; 1.0142x over previous
import jax
import jax.numpy as jnp
from jax import lax
from jax.experimental import pallas as pl
from jax.experimental.pallas import tpu as pltpu
from jax.experimental.pallas import tpu_sc as plsc

N = 10000
E = 160000
D = 256
H = 128
NC = 2
NT = 16
L = 16
EC = E // NT
CK = 80
ROWS = EC // CK
NPT = N // NT
BN = 2000
GB = N // BN
SHIFT = 14
MASK = (1 << SHIFT) - 1


def _mm_body(x_ref, w_ref, glo_ref, ghi_ref):
    xb = x_ref[...]
    glo_ref[...] = jnp.dot(xb, w_ref[:, :H], preferred_element_type=jnp.float32)
    ghi_ref[...] = jnp.dot(xb, w_ref[:, H:], preferred_element_type=jnp.float32)


def _matmul(x, W1):
    return pl.pallas_call(
        _mm_body,
        grid=(GB,),
        in_specs=[
            pl.BlockSpec((BN, D), lambda i: (i, 0)),
            pl.BlockSpec((D, D), lambda i: (0, 0)),
        ],
        out_specs=[pl.BlockSpec((BN, H), lambda i: (i, 0))] * 2,
        out_shape=[jax.ShapeDtypeStruct((N, H), jnp.float32)] * 2,
    )(x, W1)


NROW = (N + H - 1) // H


def _sc_body(idx_hbm, w_hbm, glo_hbm, ghi_hbm, fcw_hbm,
             hlo_out, hhi_out, v_out, v_out2, dinv_out,
             idx1, norm1, sidxA, didxA, sidxB, didxB, rows80,
             gbuf, gbuf2, acc_sp, deg_sp, v_sp,
             gsemA, gsemB, gsem2A, gsem2B, ssemA, ssemB):
    c = lax.axis_index("c")
    s = lax.axis_index("s")
    z16 = jnp.zeros((L,), jnp.float32)
    iota16 = lax.iota(jnp.int32, L)

    pltpu.sync_copy(idx_hbm.at[pl.ds(s * EC, EC)], idx1)
    pltpu.sync_copy(w_hbm.at[pl.ds(s * EC, EC)], norm1)

    def _zero2d(buf):
        def _z(r, _):
            for f in range(H // L):
                buf[r, pl.ds(f * L, L)] = z16
            return 0
        lax.fori_loop(0, CK, _z, 0)

    _zero2d(gbuf)
    _zero2d(gbuf2)
    for k in range(CK // L):
        rows80[pl.ds(k * L, L)] = iota16 + k * L

    @pl.when(s == 0)
    def _():
        pltpu.sync_copy(gbuf, deg_sp)
        pltpu.sync_copy(gbuf, v_sp)

    for r in range(NPT // CK):
        pltpu.sync_copy(gbuf, acc_sp.at[pl.ds(s * NPT + r * CK, CK)])
    pltpu.sync_copy(gbuf.at[pl.ds(0, NPT % CK)],
                    acc_sp.at[pl.ds(s * NPT + NPT - NPT % CK, NPT % CK)])
    plsc.subcore_barrier()

    def _deg(j, _):
        for k in range(CK // L):
            p16 = idx1[pl.ds(j * CK + k * L, L)]
            d16 = p16 & MASK
            w16 = norm1[pl.ds(j * CK + k * L, L)]
            plsc.addupdate_scatter(
                gbuf, [lax.shift_right_logical(d16, 7), d16 & (H - 1)], w16)
        return 0

    def _deg5(j5, _):
        for u in range(5):
            _deg(j5 * 5 + u, 0)
        return 0
    lax.fori_loop(0, ROWS // 5, _deg5, 0)
    pltpu.sync_copy(gbuf, deg_sp.at[rows80], add=True)
    plsc.subcore_barrier()

    pltpu.sync_copy(deg_sp, gbuf)

    def _dinv(r, _):
        for f in range(H // L):
            d = gbuf[r, pl.ds(f * L, L)] + 1.0
            i0 = jnp.int32(0x5F3759DF) - lax.shift_right_logical(
                lax.bitcast_convert_type(d, jnp.int32), 1)
            y = lax.bitcast_convert_type(i0, jnp.float32)
            y = y * (1.5 - 0.5 * d * y * y)
            y = y * (1.5 - 0.5 * d * y * y)
            y = y * (1.5 - 0.5 * d * y * y)
            gbuf[r, pl.ds(f * L, L)] = y
        return 0
    lax.fori_loop(0, CK, _dinv, 0)

    @pl.when(jnp.logical_and(c == 1, s == 0))
    def _():
        pltpu.sync_copy(gbuf, dinv_out)

    def _norm(j, _):
        for k in range(CK // L):
            sl = pl.ds(j * CK + k * L, L)
            p16 = idx1[sl]
            s16 = lax.shift_right_logical(p16, SHIFT)
            d16 = p16 & MASK
            dv_s = plsc.load_gather(
                gbuf, [lax.shift_right_logical(s16, 7), s16 & (H - 1)])
            dv_d = plsc.load_gather(
                gbuf, [lax.shift_right_logical(d16, 7), d16 & (H - 1)])
            norm1[sl] = dv_s * norm1[sl] * dv_d
        return 0

    def _norm5(j5, _):
        for u in range(5):
            _norm(j5 * 5 + u, 0)
        return 0
    lax.fori_loop(0, ROWS // 5, _norm5, 0)

    def _unpack(j, si, di):
        for k in range(CK // L):
            p16 = idx1[pl.ds(j * CK + k * L, L)]
            si[pl.ds(k * L, L)] = lax.shift_right_logical(p16, SHIFT)
            di[pl.ds(k * L, L)] = p16 & MASK

    def _scale(j, gb, k0, k1):
        def _row16(k, _):
            n16 = norm1[pl.ds(j * CK + k * L, L)]
            for r in range(L):
                e = k * L + r
                n = n16[r]
                for f in range(H // L):
                    gb[e, pl.ds(f * L, L)] = gb[e, pl.ds(f * L, L)] * n
            return 0
        lax.fori_loop(k0, k1, _row16, 0)

    GS0 = 3 * L

    def _edges(g_hbm):
        _unpack(0, sidxA, didxA)
        pltpu.async_copy(g_hbm.at[sidxA.at[pl.ds(0, GS0)]],
                         gbuf.at[pl.ds(0, GS0)], gsemA)
        pltpu.async_copy(g_hbm.at[sidxA.at[pl.ds(GS0, CK - GS0)]],
                         gbuf.at[pl.ds(GS0, CK - GS0)], gsem2A)

        def _iter(j, own, oth):
            gb, si, di, gsem, gsem2, ssem = own
            gbo, sio, dio, gsemo, gsem2o, ssemo = oth

            @pl.when(j + 1 < ROWS)
            def _():
                @pl.when(j >= 1)
                def _():
                    pltpu.make_async_copy(gbo, acc_sp.at[dio], ssemo).wait()
                _unpack(j + 1, sio, dio)
                pltpu.async_copy(g_hbm.at[sio.at[pl.ds(0, GS0)]],
                                 gbo.at[pl.ds(0, GS0)], gsemo)
                pltpu.async_copy(g_hbm.at[sio.at[pl.ds(GS0, CK - GS0)]],
                                 gbo.at[pl.ds(GS0, CK - GS0)], gsem2o)

            pltpu.make_async_copy(g_hbm.at[si.at[pl.ds(0, GS0)]],
                                  gb.at[pl.ds(0, GS0)], gsem).wait()
            _scale(j, gb, 0, GS0 // L)
            pltpu.make_async_copy(g_hbm.at[si.at[pl.ds(GS0, CK - GS0)]],
                                  gb.at[pl.ds(GS0, CK - GS0)], gsem2).wait()
            _scale(j, gb, GS0 // L, CK // L)
            pltpu.async_copy(gb, acc_sp.at[di], ssem, add=True)

        A = (gbuf, sidxA, didxA, gsemA, gsem2A, ssemA)
        B = (gbuf2, sidxB, didxB, gsemB, gsem2B, ssemB)

        def _chunk(j, _):
            @pl.when(j % 2 == 0)
            def _():
                _iter(j, A, B)

            @pl.when(j % 2 == 1)
            def _():
                _iter(j, B, A)
            return 0
        lax.fori_loop(0, ROWS, _chunk, 0)
        pltpu.make_async_copy(gbuf2, acc_sp.at[didxB], ssemB).wait()
        pltpu.make_async_copy(gbuf, acc_sp.at[didxA], ssemA).wait()

    @pl.when(c == 0)
    def _():
        _edges(glo_hbm)

    @pl.when(c == 1)
    def _():
        _edges(ghi_hbm)
    plsc.subcore_barrier()

    @pl.when(c == 0)
    def _():
        pltpu.sync_copy(acc_sp.at[pl.ds(s * NPT, NPT)], hlo_out.at[s])

    @pl.when(c == 1)
    def _():
        pltpu.sync_copy(acc_sp.at[pl.ds(s * NPT, NPT)], hhi_out.at[s])

    pltpu.sync_copy(fcw_hbm, gbuf)
    _zero2d(gbuf2)

    def _vscat(j, _):
        for k in range(CK // L):
            sl = pl.ds(j * CK + k * L, L)
            p16 = idx1[sl]
            s16 = lax.shift_right_logical(p16, SHIFT)
            d16 = p16 & MASK
            fw = plsc.load_gather(
                gbuf, [lax.shift_right_logical(d16, 7), d16 & (H - 1)])
            plsc.addupdate_scatter(
                gbuf2, [lax.shift_right_logical(s16, 7), s16 & (H - 1)],
                norm1[sl] * fw)
        return 0
    lax.fori_loop(c * (ROWS // 2), (c + 1) * (ROWS // 2) + c * (ROWS % 2),
                  _vscat, 0)
    pltpu.sync_copy(gbuf2, v_sp.at[rows80], add=True)
    plsc.subcore_barrier()

    @pl.when(jnp.logical_and(c == 0, s == 0))
    def _():
        pltpu.sync_copy(v_sp, v_out)

    @pl.when(jnp.logical_and(c == 1, s == 0))
    def _():
        pltpu.sync_copy(v_sp, v_out2)


def _sc_call(idx2, w2, g_lo, g_hi, fcw2):
    mesh = plsc.VectorSubcoreMesh(core_axis_name="c", subcore_axis_name="s")
    f = pl.kernel(
        _sc_body,
        out_type=[
            jax.ShapeDtypeStruct((NT, NPT, H), jnp.float32),
            jax.ShapeDtypeStruct((NT, NPT, H), jnp.float32),
            jax.ShapeDtypeStruct((CK, H), jnp.float32),
            jax.ShapeDtypeStruct((CK, H), jnp.float32),
            jax.ShapeDtypeStruct((CK, H), jnp.float32),
        ],
        mesh=mesh,
        compiler_params=pltpu.CompilerParams(needs_layout_passes=False),
        scratch_types=[
            pltpu.VMEM((EC,), jnp.int32),
            pltpu.VMEM((EC,), jnp.float32),
            pltpu.VMEM((CK,), jnp.int32),
            pltpu.VMEM((CK,), jnp.int32),
            pltpu.VMEM((CK,), jnp.int32),
            pltpu.VMEM((CK,), jnp.int32),
            pltpu.VMEM((CK,), jnp.int32),
            pltpu.VMEM((CK, H), jnp.float32),
            pltpu.VMEM((CK, H), jnp.float32),
            pltpu.VMEM_SHARED((N, H), jnp.float32),
            pltpu.VMEM_SHARED((CK, H), jnp.float32),
            pltpu.VMEM_SHARED((CK, H), jnp.float32),
            pltpu.SemaphoreType.DMA,
            pltpu.SemaphoreType.DMA,
            pltpu.SemaphoreType.DMA,
            pltpu.SemaphoreType.DMA,
            pltpu.SemaphoreType.DMA,
            pltpu.SemaphoreType.DMA,
        ],
    )
    return f(idx2, w2, g_lo, g_hi, fcw2)


def _red_body(hlo_ref, hhi_ref, glo_ref, ghi_ref, dinv_ref, v_ref,
              fcw_ref, b1_ref, w2_ref, b2_ref, fcb_ref, out_ref, acc, sfc):
    i = pl.program_id(0)

    @pl.when(i == 0)
    def _():
        acc[...] = jnp.zeros((2, H), jnp.float32)
        sfc[0] = 0.0

    dv = dinv_ref[0, 0]
    dv2 = dv * dv
    fw = fcw_ref[0, 0]
    vf = (v_ref[0, 0] + dv2 * fw)[None, :]
    for q, (h_ref, g_ref) in enumerate(((hlo_ref, glo_ref), (hhi_ref, ghi_ref))):
        hq = jnp.maximum(
            h_ref[...] + dv2[:, None] * g_ref[...] + b1_ref[q][None, :], 0.0)
        acc[q:q + 1, :] = acc[q:q + 1, :] + jnp.dot(
            vf, hq, preferred_element_type=jnp.float32)
    sfc[0] = sfc[0] + jnp.sum(fw)

    @pl.when(i == GB - 1)
    def _():
        total = (jnp.sum(acc[...] * w2_ref[...])
                 + b2_ref[0, 0] * sfc[0] + fcb_ref[0, 0])
        out_ref[...] = jnp.reshape(total, (1, 1))


def _reduce(hlo, hhi, g_lo, g_hi, dinv, v, fcw, b1, W2, b2, fc_b):
    dinv2d = dinv.reshape(GB, 1, BN)
    v2d = v.reshape(GB, 1, BN)
    fcw2d = fcw.reshape(GB, 1, BN)
    b1r = b1.reshape(2, H)
    w2r = W2[:, 0].reshape(2, H)
    b2r = b2.reshape(1, 1)
    fcbr = fc_b.reshape(1, 1)
    out = pl.pallas_call(
        _red_body,
        grid=(GB,),
        in_specs=(
            [pl.BlockSpec((BN, H), lambda i: (i, 0))] * 4 + [
                pl.BlockSpec((1, 1, BN), lambda i: (i, 0, 0)),
                pl.BlockSpec((1, 1, BN), lambda i: (i, 0, 0)),
                pl.BlockSpec((1, 1, BN), lambda i: (i, 0, 0)),
                pl.BlockSpec((2, H), lambda i: (0, 0)),
                pl.BlockSpec((2, H), lambda i: (0, 0)),
                pl.BlockSpec((1, 1), lambda i: (0, 0)),
                pl.BlockSpec((1, 1), lambda i: (0, 0)),
            ]
        ),
        out_specs=pl.BlockSpec((1, 1), lambda i: (0, 0)),
        out_shape=jax.ShapeDtypeStruct((1, 1), jnp.float32),
        scratch_shapes=[
            pltpu.VMEM((2, H), jnp.float32),
            pltpu.SMEM((1,), jnp.float32),
        ],
    )(hlo, hhi, g_lo, g_hi, dinv2d, v2d, fcw2d, b1r, w2r, b2r, fcbr)
    return out.reshape(1)


def kernel(x, edge_index, edge_weight, W1, b1, W2, b2, fc_W, fc_b):
    ei = edge_index.astype(jnp.int32)
    packed = jnp.bitwise_or(jnp.left_shift(ei[0], SHIFT), ei[1])
    idx2 = packed
    w2 = edge_weight
    fcw = fc_W[:, 0]
    fcw2 = jnp.pad(fcw, (0, CK * H - N)).reshape(CK, H)

    g_lo, g_hi = _matmul(x, W1)
    hlo, hhi, v2a, v2b, dinv2 = _sc_call(idx2, w2, g_lo, g_hi, fcw2)
    v = (v2a + v2b).reshape(CK * H)[:N]
    dinv = dinv2.reshape(CK * H)[:N]
    return _reduce(hlo.reshape(N, H), hhi.reshape(N, H), g_lo, g_hi,
                   dinv, v, fcw, b1, W2, b2, fc_b)
